# Initial kernel scaffold; baseline (speedup 1.0000x reference)
import jax
import jax.numpy as jnp
from jax.experimental import pallas as pl


def kernel(target_samples, X, neigh_idx, W1, W2, W, b):
    raise NotImplementedError("write your pallas kernel here")


# speedup vs baseline: 1.3157x; 1.3157x over previous
import functools

import jax
import jax.numpy as jnp
from jax import lax
from jax.experimental import pallas as pl
from jax.experimental.pallas import tpu as pltpu
from jax.experimental.pallas import tpu_sc as plsc

N_NODES = 10000
N_PAD = 10240
M = 32
D = 128
G = 64
NN = 100
NW = 32
VECS = D // 16


def _make_gather_sum(segw, seg_len, spg):
    ipg = spg * seg_len
    ng = segw // spg
    assert ipg <= 128 and segw % spg == 0
    mesh = plsc.VectorSubcoreMesh(core_axis_name="c", subcore_axis_name="s")

    @functools.partial(
        pl.kernel,
        mesh=mesh,
        out_type=jax.ShapeDtypeStruct((NW * segw, D), jnp.float32),
        scratch_types=[
            pltpu.VMEM((ng, ipg), jnp.int32),
            pltpu.VMEM((ipg, D), jnp.float32),
            pltpu.VMEM((segw, D), jnp.float32),
            pltpu.SemaphoreType.DMA,
        ],
    )
    def k(table_hbm, idx_hbm, out_hbm, idx_v, buf_v, out_v, sem):
        wid = lax.axis_index("s") * 2 + lax.axis_index("c")
        pltpu.sync_copy(idx_hbm.at[wid], idx_v)

        def gather_group(j, carry):
            pltpu.async_copy(table_hbm.at[idx_v.at[j]], buf_v, sem).wait()
            for q in range(spg):
                def seg_body(m, accs):
                    r = q * seg_len + m
                    return tuple(accs[c] + buf_v[r, pl.ds(c * 16, 16)]
                                 for c in range(VECS))
                accs = lax.fori_loop(
                    0, seg_len, seg_body,
                    tuple(jnp.zeros((16,), jnp.float32) for _ in range(VECS)))
                s = j * spg + q
                for c in range(VECS):
                    out_v[s, pl.ds(c * 16, 16)] = accs[c]
            return carry

        lax.fori_loop(0, ng, gather_group, 0)
        pltpu.sync_copy(out_v, out_hbm.at[pl.ds(wid * segw, segw)])

    return k


def _dense_layer(h, agg, wl):
    wa, wb = wl[:D], wl[D:]

    def body(xb, ab, wab, wbb, ob):
        ob[...] = jnp.maximum(
            jnp.dot(xb[...], wab[...], preferred_element_type=jnp.float32)
            + jnp.dot(ab[...], wbb[...], preferred_element_type=jnp.float32)
            * (1.0 / M),
            0.0)

    return pl.pallas_call(
        body,
        grid=(N_PAD // 512,),
        in_specs=[
            pl.BlockSpec((512, D), lambda i: (i, 0)),
            pl.BlockSpec((512, D), lambda i: (i, 0)),
            pl.BlockSpec((D, D), lambda i: (0, 0)),
            pl.BlockSpec((D, D), lambda i: (0, 0)),
        ],
        out_specs=pl.BlockSpec((512, D), lambda i: (i, 0)),
        out_shape=jax.ShapeDtypeStruct((N_PAD, D), jnp.float32),
    )(h, agg, wa, wb)


def _head(p, w_pad, b_pad):
    def body(pb, wb, bb, ob):
        y = (jnp.dot(pb[...], wb[...], preferred_element_type=jnp.float32)
             * (1.0 / NN) + bb[...])
        m = jnp.max(y, axis=1, keepdims=True)
        e = jnp.exp(y - m)
        ob[...] = y - m - jnp.log(jnp.sum(e, axis=1, keepdims=True))

    return pl.pallas_call(
        body,
        out_shape=jax.ShapeDtypeStruct((G, D), jnp.float32),
    )(p, w_pad, b_pad)


def kernel(target_samples, X, neigh_idx, W1, W2, W, b):
    X = X.astype(jnp.float32)
    idx = neigh_idx.astype(jnp.int32)
    tgt = target_samples.astype(jnp.int32)

    Xp = jnp.pad(X, ((0, N_PAD - N_NODES), (0, 0)))
    idx_p = jnp.pad(idx, ((0, N_PAD - N_NODES), (0, 0))).reshape(NW, 80, 128)
    tgt_r = tgt.reshape(NW, G // NW, NN)

    gs_neigh = _make_gather_sum(segw=N_PAD // NW, seg_len=M, spg=4)
    gs_tgt = _make_gather_sum(segw=G // NW, seg_len=NN, spg=1)

    agg1 = gs_neigh(Xp, idx_p)
    h1 = _dense_layer(Xp, agg1, W1)
    agg2 = gs_neigh(h1, idx_p)
    h2 = _dense_layer(h1, agg2, W2)
    p = gs_tgt(h2, tgt_r)

    w_pad = jnp.zeros((D, D), jnp.float32).at[:, :3].set(W)
    b_pad = jnp.full((1, D), -1e30, jnp.float32).at[0, :3].set(b)
    out = _head(p, w_pad, b_pad)
    return out[:, :3]


# speedup vs baseline: 1.4518x; 1.1034x over previous
import functools

import jax
import jax.numpy as jnp
from jax import lax
from jax.experimental import pallas as pl
from jax.experimental.pallas import tpu as pltpu
from jax.experimental.pallas import tpu_sc as plsc

N_NODES = 10000
N_PAD = 10240
M = 32
D = 128
G = 64
NN = 100
NW = 32
VECS = D // 16


def _make_gather_sum(segw, seg_len, spg):
    ipg = spg * seg_len
    ng = segw // spg
    assert ipg <= 128 and segw % spg == 0
    mesh = plsc.VectorSubcoreMesh(core_axis_name="c", subcore_axis_name="s")

    nb = 2
    assert ng % nb == 0

    @functools.partial(
        pl.kernel,
        mesh=mesh,
        out_type=jax.ShapeDtypeStruct((NW * segw, D), jnp.float32),
        scratch_types=[
            pltpu.VMEM((ng, ipg), jnp.int32),
            pltpu.VMEM((ipg, D), jnp.float32),
            pltpu.VMEM((ipg, D), jnp.float32),
            pltpu.VMEM((segw, D), jnp.float32),
            pltpu.SemaphoreType.DMA,
            pltpu.SemaphoreType.DMA,
        ],
    )
    def k(table_hbm, idx_hbm, out_hbm, idx_v, buf0, buf1, out_v, sem0, sem1):
        wid = lax.axis_index("s") * 2 + lax.axis_index("c")
        pltpu.sync_copy(idx_hbm.at[wid], idx_v)
        bufs = (buf0, buf1)
        sems = (sem0, sem1)

        for b in range(nb):
            pltpu.async_copy(table_hbm.at[idx_v.at[b]], bufs[b], sems[b])

        def gather_group(t, carry):
            for b in range(nb):
                j = t * nb + b
                buf = bufs[b]
                pltpu.make_async_copy(
                    table_hbm.at[idx_v.at[j]], buf, sems[b]).wait()
                for q in range(spg):
                    accs = [buf[q * seg_len, pl.ds(c * 16, 16)]
                            for c in range(VECS)]
                    for m in range(1, seg_len):
                        r = q * seg_len + m
                        for c in range(VECS):
                            accs[c] = accs[c] + buf[r, pl.ds(c * 16, 16)]
                    s = j * spg + q
                    for c in range(VECS):
                        out_v[s, pl.ds(c * 16, 16)] = accs[c]

                @pl.when(t < (ng // nb) - 1)
                def _():
                    pltpu.async_copy(
                        table_hbm.at[idx_v.at[j + nb]], bufs[b], sems[b])
            return carry

        lax.fori_loop(0, ng // nb, gather_group, 0)
        pltpu.sync_copy(out_v, out_hbm.at[pl.ds(wid * segw, segw)])

    return k


def _dense_layer(h, agg, wl):
    wa, wb = wl[:D], wl[D:]

    def body(xb, ab, wab, wbb, ob):
        ob[...] = jnp.maximum(
            jnp.dot(xb[...], wab[...], preferred_element_type=jnp.float32)
            + jnp.dot(ab[...], wbb[...], preferred_element_type=jnp.float32)
            * (1.0 / M),
            0.0)

    return pl.pallas_call(
        body,
        grid=(N_PAD // 512,),
        in_specs=[
            pl.BlockSpec((512, D), lambda i: (i, 0)),
            pl.BlockSpec((512, D), lambda i: (i, 0)),
            pl.BlockSpec((D, D), lambda i: (0, 0)),
            pl.BlockSpec((D, D), lambda i: (0, 0)),
        ],
        out_specs=pl.BlockSpec((512, D), lambda i: (i, 0)),
        out_shape=jax.ShapeDtypeStruct((N_PAD, D), jnp.float32),
    )(h, agg, wa, wb)


def _head(p, w_pad, b_pad):
    def body(pb, wb, bb, ob):
        y = (jnp.dot(pb[...], wb[...], preferred_element_type=jnp.float32)
             * (1.0 / NN) + bb[...])
        m = jnp.max(y, axis=1, keepdims=True)
        e = jnp.exp(y - m)
        ob[...] = y - m - jnp.log(jnp.sum(e, axis=1, keepdims=True))

    return pl.pallas_call(
        body,
        out_shape=jax.ShapeDtypeStruct((G, D), jnp.float32),
    )(p, w_pad, b_pad)


def kernel(target_samples, X, neigh_idx, W1, W2, W, b):
    X = X.astype(jnp.float32)
    idx = neigh_idx.astype(jnp.int32)
    tgt = target_samples.astype(jnp.int32)

    Xp = jnp.pad(X, ((0, N_PAD - N_NODES), (0, 0)))
    idx_p = jnp.pad(idx, ((0, N_PAD - N_NODES), (0, 0))).reshape(NW, 80, 128)
    tgt_r = tgt.reshape(NW, G // NW, NN)

    gs_neigh = _make_gather_sum(segw=N_PAD // NW, seg_len=M, spg=4)
    gs_tgt = _make_gather_sum(segw=G // NW, seg_len=NN, spg=1)

    agg1 = gs_neigh(Xp, idx_p)
    h1 = _dense_layer(Xp, agg1, W1)
    agg2 = gs_neigh(h1, idx_p)
    h2 = _dense_layer(h1, agg2, W2)
    p = gs_tgt(h2, tgt_r)

    w_pad = jnp.zeros((D, D), jnp.float32).at[:, :3].set(W)
    b_pad = jnp.full((1, D), -1e30, jnp.float32).at[0, :3].set(b)
    out = _head(p, w_pad, b_pad)
    return out[:, :3]
